# baseline (device time: 68587 ns/iter reference)
import jax
import jax.numpy as jnp
from jax import lax
from jax.experimental import pallas as pl
from jax.experimental.pallas import tpu as pltpu

N_DEV = 4
SQ = 1024
SKV_LOC = 1024
HQ = 8
DH = 128
D = 1024
SCALE = 0.08838834764831843
G = 32
B1 = 896
NB1 = SQ - B1
BR = 128
BW = 384
NBLK = SQ // BR

F32 = jnp.float32
BF16 = jnp.bfloat16


def _band_masks():
    r = lax.broadcasted_iota(jnp.int32, (BR, BW), 0)
    c = lax.broadcasted_iota(jnp.int32, (BR, BW), 1)
    m0 = ((jnp.abs(r - c) <= 128) | (c < 32)) & (r >= 32)
    m1 = ((c >= r) & (c <= r + 256)) | (c < 32)
    mg = (c >= r) & (c <= r + 256)
    m7 = c >= r + 128
    return m0, m1, mg, m7


def kernel(x, Wq, K_ext, V_ext, Wo):
    x2 = x.reshape(SQ, D)
    K2 = K_ext.reshape(SKV_LOC, HQ * DH)
    V2 = V_ext.reshape(SKV_LOC, HQ * DH)

    def body(x_ref, wq_ref, k_ref, v_ref, wo_ref, out_ref,
             acc_send, l_send, l_t, bcast_acc, bcast_lt, strip_acc, strip_l,
             bc_send_sems, bc_recv_sems, bcl_send_sems, bcl_recv_sems,
             fwd_send_sems, fwdl_send_sems, strip_send_sems,
             strip_recv_sems):
        my = lax.axis_index("i")

        def rcopy(src, dst, ssem, rsem, dev):
            return pltpu.make_async_remote_copy(
                src_ref=src, dst_ref=dst, send_sem=ssem, recv_sem=rsem,
                device_id=(dev,), device_id_type=pl.DeviceIdType.MESH)

        def wait_recv(dst, rsem):
            rcopy(dst, dst, rsem, rsem, 0).wait_recv()

        def wait_send(src, ssem):
            rcopy(src, src, ssem, ssem, 0).wait_send()

        def mm(a, b):
            return jnp.dot(a, b, preferred_element_type=F32)

        def mmT(a, b):
            return lax.dot_general(a, b, (((1,), (1,)), ((), ())),
                                   preferred_element_type=F32)

        barrier_sem = pltpu.get_barrier_semaphore()
        for off in (1, 2, 3):
            pl.semaphore_signal(barrier_sem, inc=1,
                                device_id=((my + off) % N_DEV,),
                                device_id_type=pl.DeviceIdType.MESH)
        pl.semaphore_wait(barrier_sem, N_DEV - 1)

        rows_g = pl.ds(0, G)
        rows_b1 = pl.ds(B1, NB1)
        rows_sg = pl.ds(0, G)
        rows_sb1 = pl.ds(G, NB1)

        xb = x_ref[...].astype(BF16)
        wqb = wq_ref[...].astype(BF16)
        q = mm(xb, wqb)
        q = (q * SCALE).astype(BF16)

        def wait_strips(role):
            for s in (1, 2, 3):
                if s == role:
                    continue
                wait_recv(strip_acc.at[s, rows_sg, :],
                          strip_recv_sems.at[s, 0])
                wait_recv(strip_l.at[s, rows_sg, :],
                          strip_recv_sems.at[s, 2])
                if s == 1:
                    wait_recv(strip_acc.at[s, rows_sb1, :],
                              strip_recv_sems.at[s, 1])
                    wait_recv(strip_l.at[s, rows_sb1, :],
                              strip_recv_sems.at[s, 3])

        def finalize_head(h, role, acc_h, l_h, out_val):
            sl = slice(h * DH, (h + 1) * DH)
            accg, lg = acc_h[0:G], l_h[0:G]
            for s in (1, 2, 3):
                if s == role:
                    accg = accg + acc_send[rows_g, sl].astype(F32)
                    lg = lg + l_send[rows_g, h:h + 1]
                else:
                    accg = accg + strip_acc[s, rows_sg, sl].astype(F32)
                    lg = lg + strip_l[s, rows_sg, h:h + 1]
            if role == 1:
                accb1 = acc_h[B1:SQ] + acc_send[rows_b1, sl].astype(F32)
                lb1 = l_h[B1:SQ] + l_send[rows_b1, h:h + 1]
            else:
                accb1 = acc_h[B1:SQ] + strip_acc[1, rows_sb1, sl].astype(F32)
                lb1 = l_h[B1:SQ] + strip_l[1, rows_sb1, h:h + 1]
            num = jnp.concatenate([accg, acc_h[G:B1], accb1], axis=0)
            den = jnp.concatenate([lg, l_h[G:B1], lb1], axis=0)
            ctx_h = (num / den).astype(BF16)
            wo_h = wo_ref[h * DH:(h + 1) * DH, :].astype(BF16)
            p = mm(ctx_h, wo_h)
            return p if out_val is None else out_val + p

        @pl.when(my == 0)
        def _():
            m0, m1, mg, m7 = _band_masks()
            cglob = lax.broadcasted_iota(jnp.int32, (BR, BR), 1) < 32
            out_val = None
            for h in range(HQ):
                sl = slice(h * DH, (h + 1) * DH)
                kh = k_ref[:, sl].astype(BF16)
                vh = v_ref[:, sl].astype(BF16)
                qh = q[:, sl]
                w_g = jnp.exp(mmT(qh[0:G], kh))
                l_gl = jnp.sum(w_g, axis=1, keepdims=True)
                acc_gl = mm(w_g.astype(BF16), vh)
                acc_blocks, l_blocks = [acc_gl], [l_gl]
                for b in range(NBLK):
                    w0 = min(max(0, BR * b - BR), SKV_LOC - BW)
                    mask = {0: m0, 1: m1, NBLK - 1: m7}.get(b, mg)
                    s_b = mmT(qh[BR * b:BR * b + BR], kh[w0:w0 + BW])
                    w_b = jnp.where(mask, jnp.exp(s_b), 0.0)
                    lb = jnp.sum(w_b, axis=1, keepdims=True)
                    accb = mm(w_b.astype(BF16), vh[w0:w0 + BW])
                    if b >= 2:
                        s_s = mmT(qh[BR * b:BR * b + BR], kh[0:BR])
                        w_s = jnp.where(cglob, jnp.exp(s_s), 0.0)
                        lb = lb + jnp.sum(w_s, axis=1, keepdims=True)
                        accb = accb + mm(w_s.astype(BF16), vh[0:BR])
                    if b == 0:
                        accb, lb = accb[G:BR], lb[G:BR]
                    acc_blocks.append(accb)
                    l_blocks.append(lb)
                acc_h = jnp.concatenate(acc_blocks, axis=0)
                l_h = jnp.concatenate(l_blocks, axis=0)
                acc_send[:, sl] = acc_h.astype(BF16)
                l_t[h:h + 1, :] = jnp.reshape(l_h, (1, SQ))
                for d_i, dst in ((0, 1), (1, 3)):
                    rcopy(acc_send.at[:, sl], bcast_acc.at[:, sl],
                          bc_send_sems.at[d_i, h], bc_recv_sems.at[h],
                          dst).start()
                    rcopy(l_t.at[h:h + 1, :], bcast_lt.at[h:h + 1, :],
                          bcl_send_sems.at[d_i, h], bcl_recv_sems.at[h],
                          dst).start()
                if h == 0:
                    wait_strips(0)
                out_val = finalize_head(h, 0, acc_h, l_h, out_val)
            out_ref[...] = out_val

        def strip_compute_send(src_id, dsts, with_b1):
            for h in range(HQ):
                sl = slice(h * DH, (h + 1) * DH)
                kh = k_ref[:, sl].astype(BF16)
                vh = v_ref[:, sl].astype(BF16)
                w_g = jnp.exp(mmT(q[0:G, sl], kh))
                l_send[rows_g, h:h + 1] = jnp.sum(w_g, axis=1, keepdims=True)
                acc_send[rows_g, sl] = mm(w_g.astype(BF16), vh).astype(BF16)
                if with_b1:
                    r = lax.broadcasted_iota(jnp.int32, (NB1, BR), 0)
                    c = lax.broadcasted_iota(jnp.int32, (NB1, BR), 1)
                    s_b = mmT(q[B1:SQ, sl], kh[0:BR])
                    w_b = jnp.where(c <= r, jnp.exp(s_b), 0.0)
                    l_send[rows_b1, h:h + 1] = jnp.sum(w_b, axis=1,
                                                       keepdims=True)
                    acc_send[rows_b1, sl] = mm(w_b.astype(BF16),
                                               vh[0:BR]).astype(BF16)
            for d_i, dst in enumerate(dsts):
                rcopy(acc_send.at[rows_g, :], strip_acc.at[src_id, rows_sg, :],
                      strip_send_sems.at[d_i, 0],
                      strip_recv_sems.at[src_id, 0], dst).start()
                rcopy(l_send.at[rows_g, :], strip_l.at[src_id, rows_sg, :],
                      strip_send_sems.at[d_i, 2],
                      strip_recv_sems.at[src_id, 2], dst).start()
                if with_b1:
                    rcopy(acc_send.at[rows_b1, :],
                          strip_acc.at[src_id, rows_sb1, :],
                          strip_send_sems.at[d_i, 1],
                          strip_recv_sems.at[src_id, 1], dst).start()
                    rcopy(l_send.at[rows_b1, :],
                          strip_l.at[src_id, rows_sb1, :],
                          strip_send_sems.at[d_i, 3],
                          strip_recv_sems.at[src_id, 3], dst).start()

        def recv_side(role, dsts, with_b1, fwd_parity):
            strip_compute_send(role, dsts, with_b1)
            wait_strips(role)
            out_val = None
            for h in range(HQ):
                sl = slice(h * DH, (h + 1) * DH)
                wait_recv(bcast_acc.at[:, sl], bc_recv_sems.at[h])
                wait_recv(bcast_lt.at[h:h + 1, :], bcl_recv_sems.at[h])
                if fwd_parity is not None and h % 2 == fwd_parity:
                    rcopy(bcast_acc.at[:, sl], bcast_acc.at[:, sl],
                          fwd_send_sems.at[h], bc_recv_sems.at[h], 2).start()
                    rcopy(bcast_lt.at[h:h + 1, :], bcast_lt.at[h:h + 1, :],
                          fwdl_send_sems.at[h], bcl_recv_sems.at[h],
                          2).start()
                out_val = finalize_head(
                    h, role, bcast_acc[:, sl].astype(F32),
                    jnp.reshape(bcast_lt[h:h + 1, :], (SQ, 1)), out_val)
            out_ref[...] = out_val

        @pl.when(my == 1)
        def _():
            recv_side(1, (0, 2, 3), True, 0)

        @pl.when(my == 2)
        def _():
            recv_side(2, (0, 1, 3), False, None)

        @pl.when(my == 3)
        def _():
            recv_side(3, (0, 1, 2), False, 1)

        @pl.when(my == 0)
        def _():
            for h in range(HQ):
                sl = slice(h * DH, (h + 1) * DH)
                for d_i in (0, 1):
                    wait_send(acc_send.at[:, sl], bc_send_sems.at[d_i, h])
                    wait_send(l_t.at[h:h + 1, :], bcl_send_sems.at[d_i, h])

        def drain_strips(with_b1):
            for d_i in range(3):
                wait_send(acc_send.at[rows_g, :], strip_send_sems.at[d_i, 0])
                wait_send(l_send.at[rows_g, :], strip_send_sems.at[d_i, 2])
                if with_b1:
                    wait_send(acc_send.at[rows_b1, :],
                              strip_send_sems.at[d_i, 1])
                    wait_send(l_send.at[rows_b1, :],
                              strip_send_sems.at[d_i, 3])

        def drain_fwd(parity):
            for h in range(HQ):
                if h % 2 == parity:
                    wait_send(bcast_acc.at[:, h * DH:(h + 1) * DH],
                              fwd_send_sems.at[h])
                    wait_send(bcast_lt.at[h:h + 1, :], fwdl_send_sems.at[h])

        @pl.when(my == 1)
        def _():
            drain_strips(True)
            drain_fwd(0)

        @pl.when(my == 2)
        def _():
            drain_strips(False)

        @pl.when(my == 3)
        def _():
            drain_strips(False)
            drain_fwd(1)

        def exit_barrier(second_barrier):
            for off in (1, 2, 3):
                pl.semaphore_signal(second_barrier, inc=1,
                                    device_id=((my + off) % N_DEV,),
                                    device_id_type=pl.DeviceIdType.MESH)
            pl.semaphore_wait(second_barrier, N_DEV - 1)

        pl.run_scoped(exit_barrier,
                      second_barrier=pltpu.SemaphoreType.REGULAR)

    out = pl.pallas_call(
        body,
        out_shape=jax.ShapeDtypeStruct((SQ, D), jnp.float32),
        in_specs=[pl.BlockSpec(memory_space=pltpu.VMEM)] * 5,
        out_specs=pl.BlockSpec(memory_space=pltpu.VMEM),
        scratch_shapes=[
            pltpu.VMEM((SQ, D), BF16),
            pltpu.VMEM((SQ, HQ), F32),
            pltpu.VMEM((HQ, SQ), F32),
            pltpu.VMEM((SQ, D), BF16),
            pltpu.VMEM((HQ, SQ), F32),
            pltpu.VMEM((N_DEV, G + NB1, D), BF16),
            pltpu.VMEM((N_DEV, G + NB1, HQ), F32),
            pltpu.SemaphoreType.DMA((2, HQ)),
            pltpu.SemaphoreType.DMA((HQ,)),
            pltpu.SemaphoreType.DMA((2, HQ)),
            pltpu.SemaphoreType.DMA((HQ,)),
            pltpu.SemaphoreType.DMA((HQ,)),
            pltpu.SemaphoreType.DMA((HQ,)),
            pltpu.SemaphoreType.DMA((3, 4)),
            pltpu.SemaphoreType.DMA((N_DEV, 4)),
        ],
        compiler_params=pltpu.CompilerParams(collective_id=0),
    )(x2, Wq, K2, V2, Wo)
    return out.reshape(1, SQ, D)


# device time: 57767 ns/iter; 1.1873x vs baseline; 1.1873x over previous
import jax
import jax.numpy as jnp
from jax import lax
from jax.experimental import pallas as pl
from jax.experimental.pallas import tpu as pltpu

N_DEV = 4
SQ = 1024
SKV_LOC = 1024
HQ = 8
DH = 128
D = 1024
SCALE = 0.08838834764831843
G = 32
B1 = 896
NB1 = SQ - B1
BR = 128
BW = 384
NBLK = SQ // BR

F32 = jnp.float32
BF16 = jnp.bfloat16


def _band_masks():
    r = lax.broadcasted_iota(jnp.int32, (BR, BW), 0)
    c = lax.broadcasted_iota(jnp.int32, (BR, BW), 1)
    m0 = ((jnp.abs(r - c) <= 128) | (c < 32)) & (r >= 32)
    m1 = ((c >= r) & (c <= r + 256)) | (c < 32)
    mg = (c >= r) & (c <= r + 256)
    m7 = c >= r + 128
    return m0, m1, mg, m7


def kernel(x, Wq, K_ext, V_ext, Wo):
    x2 = x.reshape(SQ, D)
    K2 = K_ext.reshape(SKV_LOC, HQ * DH)
    V2 = V_ext.reshape(SKV_LOC, HQ * DH)

    def body(x_ref, wq_ref, k_ref, v_ref, wo_ref, out_ref,
             acc_send, l_send, bcast_acc, bcast_lg, strip_acc, strip_l,
             bc_send_sems, bc_recv_sems, bcl_send_sems, bcl_recv_sems,
             fwd_send_sems, fwdl_send_sems, strip_send_sems,
             strip_recv_sems):
        my = lax.axis_index("i")

        def rcopy(src, dst, ssem, rsem, dev):
            return pltpu.make_async_remote_copy(
                src_ref=src, dst_ref=dst, send_sem=ssem, recv_sem=rsem,
                device_id=(dev,), device_id_type=pl.DeviceIdType.MESH)

        def wait_recv(dst, rsem):
            rcopy(dst, dst, rsem, rsem, 0).wait_recv()

        def wait_send(src, ssem):
            rcopy(src, src, ssem, ssem, 0).wait_send()

        def mm(a, b):
            return jnp.dot(a, b, preferred_element_type=F32)

        def mmT(a, b):
            return lax.dot_general(a, b, (((1,), (1,)), ((), ())),
                                   preferred_element_type=F32)

        barrier_sem = pltpu.get_barrier_semaphore()
        for off in (1, 2, 3):
            pl.semaphore_signal(barrier_sem, inc=1,
                                device_id=((my + off) % N_DEV,),
                                device_id_type=pl.DeviceIdType.MESH)
        pl.semaphore_wait(barrier_sem, N_DEV - 1)

        rows_g = pl.ds(0, G)
        rows_b1 = pl.ds(B1, NB1)
        rows_sg = pl.ds(0, G)
        rows_sb1 = pl.ds(G, NB1)

        xb = x_ref[...].astype(BF16)
        wqb = wq_ref[...].astype(BF16)
        q = mm(xb, wqb)
        q = (q * SCALE).astype(BF16)

        def wait_strips(role):
            for s in (1, 2, 3):
                if s == role:
                    continue
                wait_recv(strip_acc.at[s, rows_sg, :],
                          strip_recv_sems.at[s, 0])
                wait_recv(strip_l.at[s, rows_sg, :],
                          strip_recv_sems.at[s, 2])
                if s == 1:
                    wait_recv(strip_acc.at[s, rows_sb1, :],
                              strip_recv_sems.at[s, 1])
                    wait_recv(strip_l.at[s, rows_sb1, :],
                              strip_recv_sems.at[s, 3])

        def merge_g(role, h, acc0_g, l0_g):
            sl = slice(h * DH, (h + 1) * DH)
            accg, lg = acc0_g, l0_g
            for s in (1, 2, 3):
                if s == role:
                    accg = accg + acc_send[rows_g, sl].astype(F32)
                    lg = lg + l_send[rows_g, h:h + 1]
                else:
                    accg = accg + strip_acc[s, rows_sg, sl].astype(F32)
                    lg = lg + strip_l[s, rows_sg, h:h + 1]
            return accg / lg

        def merge_b1(role, h, acc0_b1, l0_b1):
            sl = slice(h * DH, (h + 1) * DH)
            if role == 1:
                accb1 = acc0_b1 + acc_send[rows_b1, sl].astype(F32)
                lb1 = l0_b1 + l_send[rows_b1, h:h + 1]
            else:
                accb1 = acc0_b1 + strip_acc[1, rows_sb1, sl].astype(F32)
                lb1 = l0_b1 + strip_l[1, rows_sb1, h:h + 1]
            return accb1 / lb1

        @pl.when(my == 0)
        def _():
            m0, m1, mg, m7 = _band_masks()
            cglob = lax.broadcasted_iota(jnp.int32, (BR, BR), 1) < 32
            for h in range(HQ):
                sl = slice(h * DH, (h + 1) * DH)
                kh = k_ref[:, sl].astype(BF16)
                vh = v_ref[:, sl].astype(BF16)
                qh = q[:, sl]
                w_g = jnp.exp(mmT(qh[0:G], kh))
                l_gl = jnp.sum(w_g, axis=1, keepdims=True)
                acc_gl = mm(w_g.astype(BF16), vh)
                acc_blocks, l_blocks = [acc_gl], [l_gl]
                for b in range(NBLK):
                    w0 = min(max(0, BR * b - BR), SKV_LOC - BW)
                    mask = {0: m0, 1: m1, NBLK - 1: m7}.get(b, mg)
                    s_b = mmT(qh[BR * b:BR * b + BR], kh[w0:w0 + BW])
                    w_b = jnp.where(mask, jnp.exp(s_b), 0.0)
                    lb = jnp.sum(w_b, axis=1, keepdims=True)
                    accb = mm(w_b.astype(BF16), vh[w0:w0 + BW])
                    if b >= 2:
                        s_s = mmT(qh[BR * b:BR * b + BR], kh[0:BR])
                        w_s = jnp.where(cglob, jnp.exp(s_s), 0.0)
                        lb = lb + jnp.sum(w_s, axis=1, keepdims=True)
                        accb = accb + mm(w_s.astype(BF16), vh[0:BR])
                    if b == 0:
                        accb, lb = accb[G:BR], lb[G:BR]
                    acc_blocks.append(accb)
                    l_blocks.append(lb)
                acc_h = jnp.concatenate(acc_blocks, axis=0)
                l_h = jnp.concatenate(l_blocks, axis=0)
                ctx_mid = acc_h[G:B1] / l_h[G:B1]
                acc_send[:, sl] = jnp.concatenate(
                    [acc_h[0:G], ctx_mid, acc_h[B1:SQ]], axis=0).astype(BF16)
                l_send[:, h:h + 1] = l_h
                for d_i, dst in ((0, 1), (1, 3)):
                    rcopy(acc_send.at[:, sl], bcast_acc.at[:, sl],
                          bc_send_sems.at[d_i, h], bc_recv_sems.at[h],
                          dst).start()
                if h == 0:
                    wait_strips(0)
                ctx_g = merge_g(0, h, acc_h[0:G], l_h[0:G])
                ctx_b1 = merge_b1(0, h, acc_h[B1:SQ], l_h[B1:SQ])
                bcast_acc[:, sl] = jnp.concatenate(
                    [ctx_g, ctx_mid, ctx_b1], axis=0).astype(BF16)
            for d_i, dst in ((0, 1), (1, 3)):
                rcopy(l_send.at[rows_g, :], bcast_lg.at[pl.ds(0, G), :],
                      bcl_send_sems.at[d_i, 0], bcl_recv_sems.at[0],
                      dst).start()
                rcopy(l_send.at[rows_b1, :], bcast_lg.at[pl.ds(G, NB1), :],
                      bcl_send_sems.at[d_i, 1], bcl_recv_sems.at[1],
                      dst).start()

        def strip_compute_send(src_id, dsts, with_b1):
            for h in range(HQ):
                sl = slice(h * DH, (h + 1) * DH)
                kh = k_ref[:, sl].astype(BF16)
                vh = v_ref[:, sl].astype(BF16)
                w_g = jnp.exp(mmT(q[0:G, sl], kh))
                l_send[rows_g, h:h + 1] = jnp.sum(w_g, axis=1, keepdims=True)
                acc_send[rows_g, sl] = mm(w_g.astype(BF16), vh).astype(BF16)
                if with_b1:
                    r = lax.broadcasted_iota(jnp.int32, (NB1, BR), 0)
                    c = lax.broadcasted_iota(jnp.int32, (NB1, BR), 1)
                    s_b = mmT(q[B1:SQ, sl], kh[0:BR])
                    w_b = jnp.where(c <= r, jnp.exp(s_b), 0.0)
                    l_send[rows_b1, h:h + 1] = jnp.sum(w_b, axis=1,
                                                       keepdims=True)
                    acc_send[rows_b1, sl] = mm(w_b.astype(BF16),
                                               vh[0:BR]).astype(BF16)
            for d_i, dst in enumerate(dsts):
                rcopy(acc_send.at[rows_g, :], strip_acc.at[src_id, rows_sg, :],
                      strip_send_sems.at[d_i, 0],
                      strip_recv_sems.at[src_id, 0], dst).start()
                rcopy(l_send.at[rows_g, :], strip_l.at[src_id, rows_sg, :],
                      strip_send_sems.at[d_i, 2],
                      strip_recv_sems.at[src_id, 2], dst).start()
                if with_b1:
                    rcopy(acc_send.at[rows_b1, :],
                          strip_acc.at[src_id, rows_sb1, :],
                          strip_send_sems.at[d_i, 1],
                          strip_recv_sems.at[src_id, 1], dst).start()
                    rcopy(l_send.at[rows_b1, :],
                          strip_l.at[src_id, rows_sb1, :],
                          strip_send_sems.at[d_i, 3],
                          strip_recv_sems.at[src_id, 3], dst).start()

        def recv_side(role, dsts, with_b1, fwd_parity):
            strip_compute_send(role, dsts, with_b1)
            wait_strips(role)
            for h in range(HQ):
                sl = slice(h * DH, (h + 1) * DH)
                wait_recv(bcast_acc.at[:, sl], bc_recv_sems.at[h])
                if fwd_parity is not None and h % 2 == fwd_parity:
                    rcopy(bcast_acc.at[:, sl], bcast_acc.at[:, sl],
                          fwd_send_sems.at[h], bc_recv_sems.at[h], 2).start()
            if fwd_parity == 0:
                wait_recv(bcast_lg.at[pl.ds(0, G), :], bcl_recv_sems.at[0])
                rcopy(bcast_lg.at[pl.ds(0, G), :], bcast_lg.at[pl.ds(0, G), :],
                      fwdl_send_sems.at[0], bcl_recv_sems.at[0], 2).start()
                wait_recv(bcast_lg.at[pl.ds(G, NB1), :], bcl_recv_sems.at[1])
                rcopy(bcast_lg.at[pl.ds(G, NB1), :],
                      bcast_lg.at[pl.ds(G, NB1), :],
                      fwdl_send_sems.at[1], bcl_recv_sems.at[1], 2).start()
            else:
                wait_recv(bcast_lg.at[pl.ds(0, G), :], bcl_recv_sems.at[0])
                wait_recv(bcast_lg.at[pl.ds(G, NB1), :], bcl_recv_sems.at[1])
            if fwd_parity is not None:
                for h in range(HQ):
                    if h % 2 == fwd_parity:
                        wait_send(bcast_acc.at[:, h * DH:(h + 1) * DH],
                                  fwd_send_sems.at[h])
            for h in range(HQ):
                sl = slice(h * DH, (h + 1) * DH)
                ctx_g = merge_g(role, h, bcast_acc[rows_g, sl].astype(F32),
                                bcast_lg[pl.ds(0, G), h:h + 1])
                ctx_b1 = merge_b1(role, h, bcast_acc[rows_b1, sl].astype(F32),
                                  bcast_lg[pl.ds(G, NB1), h:h + 1])
                bcast_acc[rows_g, sl] = ctx_g.astype(BF16)
                bcast_acc[rows_b1, sl] = ctx_b1.astype(BF16)

        @pl.when(my == 1)
        def _():
            recv_side(1, (0, 2, 3), True, 0)

        @pl.when(my == 2)
        def _():
            recv_side(2, (0, 1, 3), False, None)

        @pl.when(my == 3)
        def _():
            recv_side(3, (0, 1, 2), False, 1)

        out_ref[...] = mm(bcast_acc[...], wo_ref[...].astype(BF16))

        @pl.when(my == 0)
        def _():
            for h in range(HQ):
                sl = slice(h * DH, (h + 1) * DH)
                for d_i in (0, 1):
                    wait_send(acc_send.at[:, sl], bc_send_sems.at[d_i, h])
            for d_i in (0, 1):
                wait_send(l_send.at[rows_g, :], bcl_send_sems.at[d_i, 0])
                wait_send(l_send.at[rows_b1, :], bcl_send_sems.at[d_i, 1])

        def drain_strips(with_b1):
            for d_i in range(3):
                wait_send(acc_send.at[rows_g, :], strip_send_sems.at[d_i, 0])
                wait_send(l_send.at[rows_g, :], strip_send_sems.at[d_i, 2])
                if with_b1:
                    wait_send(acc_send.at[rows_b1, :],
                              strip_send_sems.at[d_i, 1])
                    wait_send(l_send.at[rows_b1, :],
                              strip_send_sems.at[d_i, 3])

        @pl.when(my == 1)
        def _():
            drain_strips(True)
            wait_send(bcast_lg.at[pl.ds(0, G), :], fwdl_send_sems.at[0])
            wait_send(bcast_lg.at[pl.ds(G, NB1), :], fwdl_send_sems.at[1])

        @pl.when(my == 2)
        def _():
            drain_strips(False)

        @pl.when(my == 3)
        def _():
            drain_strips(False)

        def exit_barrier(second_barrier):
            for off in (1, 2, 3):
                pl.semaphore_signal(second_barrier, inc=1,
                                    device_id=((my + off) % N_DEV,),
                                    device_id_type=pl.DeviceIdType.MESH)
            pl.semaphore_wait(second_barrier, N_DEV - 1)

        pl.run_scoped(exit_barrier,
                      second_barrier=pltpu.SemaphoreType.REGULAR)

    out = pl.pallas_call(
        body,
        out_shape=jax.ShapeDtypeStruct((SQ, D), jnp.float32),
        in_specs=[pl.BlockSpec(memory_space=pltpu.VMEM)] * 5,
        out_specs=pl.BlockSpec(memory_space=pltpu.VMEM),
        scratch_shapes=[
            pltpu.VMEM((SQ, D), BF16),
            pltpu.VMEM((SQ, HQ), F32),
            pltpu.VMEM((SQ, D), BF16),
            pltpu.VMEM((G + NB1, HQ), F32),
            pltpu.VMEM((N_DEV, G + NB1, D), BF16),
            pltpu.VMEM((N_DEV, G + NB1, HQ), F32),
            pltpu.SemaphoreType.DMA((2, HQ)),
            pltpu.SemaphoreType.DMA((HQ,)),
            pltpu.SemaphoreType.DMA((2, 2)),
            pltpu.SemaphoreType.DMA((2,)),
            pltpu.SemaphoreType.DMA((HQ,)),
            pltpu.SemaphoreType.DMA((2,)),
            pltpu.SemaphoreType.DMA((3, 4)),
            pltpu.SemaphoreType.DMA((N_DEV, 4)),
        ],
        compiler_params=pltpu.CompilerParams(collective_id=0),
    )(x2, Wq, K2, V2, Wo)
    return out.reshape(1, SQ, D)
